# baseline (speedup 1.0000x reference)
import functools

import jax
import jax.numpy as jnp
from jax import lax
from jax.experimental import pallas as pl
from jax.experimental.pallas import tpu as pltpu
from jax.experimental.pallas import tpu_sc as plsc

EMBED_DIM = 64
BATCH = 16384


@functools.cache
def _build():
    info = plsc.get_sparse_core_info()
    NC, NS = info.num_cores, info.num_subcores
    NW = NC * NS
    b_per_w = BATCH // NW
    half = b_per_w // 2
    mesh = plsc.VectorSubcoreMesh(core_axis_name="c", subcore_axis_name="s")

    out_t = jax.ShapeDtypeStruct((BATCH, EMBED_DIM), jnp.float32)

    @functools.partial(
        pl.kernel,
        mesh=mesh,
        out_type=[out_t, out_t, out_t],
        scratch_types=[
            pltpu.VMEM((b_per_w,), jnp.int32),
            pltpu.VMEM((b_per_w,), jnp.int32),
            pltpu.VMEM((b_per_w, 2 * EMBED_DIM), jnp.float32),
            pltpu.VMEM((half, EMBED_DIM), jnp.float32),
            pltpu.SemaphoreType.DMA,
        ],
    )
    def lookup(W_u, W_i, W_c, id_u, id_i, id_c, out_u, out_i, out_c,
               idx_v, pair_v, buf, obuf, sem):
        wid = lax.axis_index("s") * NC + lax.axis_index("c")
        base = wid * b_per_w

        for W2, ids, out in ((W_u, id_u, out_u),
                             (W_i, id_i, out_i),
                             (W_c, id_c, out_c)):
            pltpu.sync_copy(ids.at[pl.ds(base, b_per_w)], idx_v)

            def pairs(g, _):
                pair_v[pl.ds(g * 16, 16)] = lax.shift_right_logical(
                    idx_v[pl.ds(g * 16, 16)], 1
                )
                return _

            lax.fori_loop(0, b_per_w // 16, pairs, 0, unroll=4)
            pltpu.async_copy(W2.at[pair_v], buf, sem).wait()

            for ch in range(2):
                def extract(g, _, ch=ch):
                    v = idx_v[pl.ds(ch * half + g * 16, 16)]
                    for l in range(16):
                        off = lax.mul(
                            lax.bitwise_and(v[l], 1), EMBED_DIM
                        )
                        j = ch * half + g * 16 + l
                        for k in range(EMBED_DIM // 16):
                            obuf[g * 16 + l, pl.ds(16 * k, 16)] = (
                                buf[j, pl.ds(off + 16 * k, 16)]
                            )
                    return _

                lax.fori_loop(0, half // 16, extract, 0)
                pltpu.sync_copy(
                    obuf, out.at[pl.ds(base + ch * half, half)]
                )

    return lookup


def kernel(W_user, W_item, W_category, user_id, item_id, category_id):
    lookup = _build()
    e_user, e_item, e_category = lookup(
        W_user.reshape(-1, 2 * EMBED_DIM),
        W_item.reshape(-1, 2 * EMBED_DIM),
        W_category.reshape(-1, 2 * EMBED_DIM),
        user_id.astype(jnp.int32),
        item_id.astype(jnp.int32),
        category_id.astype(jnp.int32),
    )
    return (e_user, e_item, e_category)


# speedup vs baseline: 1.0279x; 1.0279x over previous
import functools

import jax
import jax.numpy as jnp
from jax import lax
from jax.experimental import pallas as pl
from jax.experimental.pallas import tpu as pltpu
from jax.experimental.pallas import tpu_sc as plsc

EMBED_DIM = 64
BATCH = 16384


@functools.cache
def _build():
    info = plsc.get_sparse_core_info()
    NC, NS = info.num_cores, info.num_subcores
    NW = NC * NS
    b_per_w = BATCH // NW
    half = b_per_w // 2
    mesh = plsc.VectorSubcoreMesh(core_axis_name="c", subcore_axis_name="s")

    out_t = jax.ShapeDtypeStruct((BATCH, EMBED_DIM), jnp.float32)

    @functools.partial(
        pl.kernel,
        mesh=mesh,
        out_type=out_t,
        scratch_types=[
            pltpu.VMEM((b_per_w,), jnp.int32),
            pltpu.VMEM((b_per_w,), jnp.int32),
            pltpu.VMEM((b_per_w, 2 * EMBED_DIM), jnp.float32),
            pltpu.VMEM((half, EMBED_DIM), jnp.float32),
            pltpu.SemaphoreType.DMA,
        ],
    )
    def lookup(W2, ids, out, idx_v, pair_v, buf, obuf, sem):
        wid = lax.axis_index("s") * NC + lax.axis_index("c")
        base = wid * b_per_w

        pltpu.sync_copy(ids.at[pl.ds(base, b_per_w)], idx_v)

        def pairs(g, _):
            pair_v[pl.ds(g * 16, 16)] = lax.shift_right_logical(
                idx_v[pl.ds(g * 16, 16)], 1
            )
            return _

        lax.fori_loop(0, b_per_w // 16, pairs, 0, unroll=4)
        pltpu.async_copy(W2.at[pair_v], buf, sem).wait()

        for ch in range(2):
            def extract(g, _, ch=ch):
                v = idx_v[pl.ds(ch * half + g * 16, 16)]
                for l in range(16):
                    off = lax.mul(lax.bitwise_and(v[l], 1), EMBED_DIM)
                    j = ch * half + g * 16 + l
                    for k in range(EMBED_DIM // 16):
                        obuf[g * 16 + l, pl.ds(16 * k, 16)] = (
                            buf[j, pl.ds(off + 16 * k, 16)]
                        )
                return _

            lax.fori_loop(0, half // 16, extract, 0)
            pltpu.sync_copy(obuf, out.at[pl.ds(base + ch * half, half)])

    return lookup


def kernel(W_user, W_item, W_category, user_id, item_id, category_id):
    lookup = _build()
    e_user = lookup(W_user.reshape(-1, 2 * EMBED_DIM),
                    user_id.astype(jnp.int32))
    e_item = lookup(W_item.reshape(-1, 2 * EMBED_DIM),
                    item_id.astype(jnp.int32))
    e_category = lookup(W_category.reshape(-1, 2 * EMBED_DIM),
                        category_id.astype(jnp.int32))
    return (e_user, e_item, e_category)


# speedup vs baseline: 1.9582x; 1.9051x over previous
import functools

import jax
import jax.numpy as jnp
from jax import lax
from jax.experimental import pallas as pl
from jax.experimental.pallas import tpu as pltpu
from jax.experimental.pallas import tpu_sc as plsc

EMBED_DIM = 64
BATCH = 16384
CHUNK = 64


@functools.cache
def _build():
    info = plsc.get_sparse_core_info()
    NC, NS = info.num_cores, info.num_subcores
    NW = NC * NS
    b_per_w = BATCH // NW
    n_chunks = b_per_w // CHUNK
    mesh = plsc.VectorSubcoreMesh(core_axis_name="c", subcore_axis_name="s")

    out_t = jax.ShapeDtypeStruct((BATCH, EMBED_DIM), jnp.float32)

    @functools.partial(
        pl.kernel,
        mesh=mesh,
        out_type=out_t,
        scratch_types=[
            pltpu.VMEM((b_per_w,), jnp.int32),
            pltpu.VMEM((CHUNK, 8, EMBED_DIM), jnp.float32),
            pltpu.VMEM((CHUNK, EMBED_DIM), jnp.float32),
            pltpu.SemaphoreType.DMA,
        ],
    )
    def lookup(W3, ids, out, idx_v, tiles, obuf, sem):
        wid = lax.axis_index("s") * NC + lax.axis_index("c")
        base = wid * b_per_w

        pltpu.sync_copy(ids.at[pl.ds(base, b_per_w)], idx_v)

        def chunk_body(c, _):
            for g in range(CHUNK // 16):
                v = idx_v[pl.ds(c * CHUNK + g * 16, 16)]
                for l in range(16):
                    t = lax.shift_right_logical(v[l], 3)
                    pltpu.async_copy(W3.at[t], tiles.at[g * 16 + l], sem)

            pltpu.make_async_copy(
                W3.at[pl.ds(0, CHUNK)], tiles, sem
            ).wait()

            for g in range(CHUNK // 16):
                rv = lax.bitwise_and(idx_v[pl.ds(c * CHUNK + g * 16, 16)], 7)
                for l in range(16):
                    r = rv[l]
                    for k in range(EMBED_DIM // 16):
                        obuf[g * 16 + l, pl.ds(16 * k, 16)] = (
                            tiles[g * 16 + l, r, pl.ds(16 * k, 16)]
                        )

            pltpu.sync_copy(obuf, out.at[pl.ds(base + c * CHUNK, CHUNK)])
            return _

        lax.fori_loop(0, n_chunks, chunk_body, 0)

    return lookup


def kernel(W_user, W_item, W_category, user_id, item_id, category_id):
    lookup = _build()
    e_user = lookup(W_user.reshape(-1, 8, EMBED_DIM),
                    user_id.astype(jnp.int32))
    e_item = lookup(W_item.reshape(-1, 8, EMBED_DIM),
                    item_id.astype(jnp.int32))
    e_category = lookup(W_category.reshape(-1, 8, EMBED_DIM),
                        category_id.astype(jnp.int32))
    return (e_user, e_item, e_category)
